# Initial kernel scaffold; baseline (speedup 1.0000x reference)
import jax
import jax.numpy as jnp
from jax.experimental import pallas as pl


def kernel(x, conv_w, conv_b):
    raise NotImplementedError("write your pallas kernel here")


# speedup vs baseline: 2.3851x; 2.3851x over previous
import numpy as np
import jax
import jax.numpy as jnp
from jax.experimental import pallas as pl
from jax.experimental.pallas import tpu as pltpu

_ALL_ANCHORS = np.array(
    [[10, 13], [16, 30], [33, 23], [30, 61], [62, 45], [59, 119],
     [116, 90], [156, 198], [373, 326]], dtype=np.float32)
_ANCHOR_IDXS = np.array([0, 1, 2])
_NCLS = 80
_A = 3
_CH = 5 + _NCLS
_C_OUT = _A * _CH
_C_IN = 128
_N = 64
_HW = _N * _N
_STRIDE = 8.0

_o = np.arange(_C_OUT)
_k = _o % _CH
_M_SIG = np.where(_k < 2, _STRIDE, np.where(_k < 4, 0.0, 1.0)).astype(np.float32)
_M_EXP = np.zeros(_C_OUT, np.float32)
_anch = _ALL_ANCHORS[_ANCHOR_IDXS]
for _a in range(_A):
    _M_EXP[_a * _CH + 2] = _anch[_a, 0]
    _M_EXP[_a * _CH + 3] = _anch[_a, 1]
_M_X = np.where(_k == 0, _STRIDE, 0.0).astype(np.float32)
_M_Y = np.where(_k == 1, _STRIDE, 0.0).astype(np.float32)


def _yolo_kernel(x_ref, w_ref, b_ref, msig_ref, mexp_ref, mx_ref, my_ref,
                 out_ref):
    xb = x_ref[0]
    w = w_ref[...]
    y = jax.lax.dot_general(
        xb, w, (((0,), (1,)), ((), ())),
        preferred_element_type=jnp.float32)
    y = y + b_ref[...]
    sig = jax.nn.sigmoid(y)
    mexp = mexp_ref[...]
    ex = jnp.exp(jnp.where(mexp != 0.0, y, 0.0)) * mexp
    row = jax.lax.broadcasted_iota(jnp.int32, (_HW, 1), 0)
    wcol = (row & (_N - 1)).astype(jnp.float32)
    hcol = (row >> 6).astype(jnp.float32)
    res = sig * msig_ref[...] + ex + wcol * mx_ref[...] + hcol * my_ref[...]
    for a in range(_A):
        out_ref[0, a] = res[:, _CH * a:_CH * (a + 1)]


def kernel(x, conv_w, conv_b):
    B = x.shape[0]
    xf = x.reshape(B, _C_IN, _HW)
    w = conv_w[:, :, 0, 0]
    b = conv_b.reshape(1, _C_OUT)
    msig = jnp.asarray(_M_SIG).reshape(1, _C_OUT)
    mexp = jnp.asarray(_M_EXP).reshape(1, _C_OUT)
    mx = jnp.asarray(_M_X).reshape(1, _C_OUT)
    my = jnp.asarray(_M_Y).reshape(1, _C_OUT)

    out = pl.pallas_call(
        _yolo_kernel,
        grid=(B,),
        in_specs=[
            pl.BlockSpec((1, _C_IN, _HW), lambda i: (i, 0, 0)),
            pl.BlockSpec((_C_OUT, _C_IN), lambda i: (0, 0)),
            pl.BlockSpec((1, _C_OUT), lambda i: (0, 0)),
            pl.BlockSpec((1, _C_OUT), lambda i: (0, 0)),
            pl.BlockSpec((1, _C_OUT), lambda i: (0, 0)),
            pl.BlockSpec((1, _C_OUT), lambda i: (0, 0)),
            pl.BlockSpec((1, _C_OUT), lambda i: (0, 0)),
        ],
        out_specs=pl.BlockSpec((1, _A, _HW, _CH), lambda i: (i, 0, 0, 0)),
        out_shape=jax.ShapeDtypeStruct((B, _A, _HW, _CH), jnp.float32),
        compiler_params=pltpu.CompilerParams(
            dimension_semantics=("arbitrary",)),
    )(xf, w, b, msig, mexp, mx, my)
    return out.reshape(B, _A * _HW, _CH)
